# Initial kernel scaffold; baseline (speedup 1.0000x reference)
import jax
import jax.numpy as jnp
from jax.experimental import pallas as pl


def kernel(x, w_conv, b_conv, w_lin, b_lin):
    raise NotImplementedError("write your pallas kernel here")


# speedup vs baseline: 10.3508x; 10.3508x over previous
import functools

import jax
import jax.numpy as jnp
from jax import lax
from jax.experimental import pallas as pl
from jax.experimental.pallas import tpu as pltpu

H, W = 28, 28
KH, KW = 3, 3
OH, OW = H - KH + 1, W - KW + 1
C_OUT = 32
OC = OW * C_OUT
RK = KH * W
N_CLASSES = 10
TN = 512


def _net_kernel(x_ref, w2_ref, bc_ref, wl_ref, bl_ref, out_ref):
    tn = out_ref.shape[1 - 1]
    acc = jnp.zeros((tn, N_CLASSES), jnp.float32)
    for oi in range(OH):
        pat = jnp.concatenate(
            [x_ref[oi], x_ref[oi + 1], x_ref[oi + 2]], axis=1)
        a = jnp.dot(pat, w2_ref[...],
                    preferred_element_type=jnp.float32)
        ab = jnp.maximum(a + bc_ref[...], 0.0).astype(jnp.bfloat16)
        acc = acc + lax.dot_general(
            ab, wl_ref[oi],
            dimension_numbers=(((1,), (1,)), ((), ())),
            preferred_element_type=jnp.float32)
    out_ref[...] = acc + bl_ref[...]


@functools.partial(jax.jit, static_argnames=("tn",))
def _forward(x, w_conv, b_conv, w_lin, b_lin, *, tn=TN):
    n = x.shape[0]
    tn = min(tn, max(8, pl.cdiv(n, 8) * 8))
    n_tiles = pl.cdiv(n, tn)
    n_pad = n_tiles * tn

    x2 = x[:, 0, :, :]
    if n_pad != n:
        x2 = jnp.pad(x2, ((0, n_pad - n), (0, 0), (0, 0)))
    xt = jnp.transpose(x2, (1, 0, 2)).astype(jnp.bfloat16)

    wt = jnp.transpose(w_conv[:, 0, :, :], (1, 2, 0))
    eye = jnp.stack([jnp.eye(W, OW, k=-dj, dtype=w_conv.dtype)
                     for dj in range(KW)])
    w2 = jnp.einsum("jko,djc->dkoc", eye, wt).reshape(RK, OC)
    w2 = w2.astype(jnp.bfloat16)
    bc = jnp.tile(b_conv.astype(jnp.float32), OW).reshape(1, OC)

    wl = (w_lin.reshape(N_CLASSES, C_OUT, OH, OW)
               .transpose(2, 0, 3, 1)
               .reshape(OH, N_CLASSES, OC)).astype(jnp.bfloat16)
    bl = b_lin.reshape(1, N_CLASSES).astype(jnp.float32)

    out = pl.pallas_call(
        _net_kernel,
        out_shape=jax.ShapeDtypeStruct((n_pad, N_CLASSES), jnp.float32),
        grid=(n_tiles,),
        in_specs=[
            pl.BlockSpec((H, tn, W), lambda i: (0, i, 0)),
            pl.BlockSpec((RK, OC), lambda i: (0, 0)),
            pl.BlockSpec((1, OC), lambda i: (0, 0)),
            pl.BlockSpec((OH, N_CLASSES, OC), lambda i: (0, 0, 0)),
            pl.BlockSpec((1, N_CLASSES), lambda i: (0, 0)),
        ],
        out_specs=pl.BlockSpec((tn, N_CLASSES), lambda i: (i, 0)),
        compiler_params=pltpu.CompilerParams(
            dimension_semantics=("parallel",),
            vmem_limit_bytes=64 << 20),
    )(xt, w2, bc, wl, bl)
    return out[:n]


def kernel(x, w_conv, b_conv, w_lin, b_lin):
    return _forward(x, w_conv, b_conv, w_lin, b_lin)
